# Initial kernel scaffold; baseline (speedup 1.0000x reference)
import jax
import jax.numpy as jnp
from jax.experimental import pallas as pl


def kernel(x, w_gate, w_up, w_down, w_up_shared, w_down_shared):
    raise NotImplementedError("write your pallas kernel here")


# speedup vs baseline: 1.8770x; 1.8770x over previous
import functools

import jax
import jax.numpy as jnp
from jax import lax
from jax.experimental import pallas as pl
from jax.experimental.pallas import tpu as pltpu
from jax.experimental.pallas import tpu_sc as plsc

DIM = 2048
NE = 8
HID = 2688
T = 2048
BLK = 128
R = 5120
NB = R // BLK
TB = T // BLK
NCHUNK = 10
CHUNK = R // NCHUNK

_DEF = lax.Precision.DEFAULT
_HI = lax.Precision.HIGHEST


def _fiota(shape, dim):
    return lax.broadcasted_iota(jnp.int32, shape, dim).astype(jnp.float32)


def _r1_body(x_ref, wg_ref, m1_ref, m2_ref, w1_ref, w2_ref):
    x = x_ref[...]
    wg = wg_ref[...]
    logits = lax.dot_general(x, wg, (((1,), (1,)), ((), ())),
                             precision=_DEF,
                             preferred_element_type=jnp.float32)
    scores = jax.nn.sigmoid(logits)
    iota8 = _fiota((BLK, NE), 1)
    s1 = jnp.max(scores, axis=1, keepdims=True)
    e1 = jnp.min(jnp.where(scores >= s1, iota8, float(NE)), axis=1,
                 keepdims=True)
    m1 = (iota8 == e1)
    masked = jnp.where(m1, -1.0, scores)
    s2 = jnp.max(masked, axis=1, keepdims=True)
    e2 = jnp.min(jnp.where(masked >= s2, iota8, float(NE)), axis=1,
                 keepdims=True)
    m2 = (iota8 == e2)
    m1_ref[...] = m1.astype(jnp.float32)
    m2_ref[...] = m2.astype(jnp.float32)
    w1_ref[...] = s1 * s1
    w2_ref[...] = s2 * s2


def _router_select(x2):
    return pl.pallas_call(
        _r1_body,
        grid=(TB,),
        in_specs=[
            pl.BlockSpec((BLK, DIM), lambda b: (b, 0)),
            pl.BlockSpec((NE, DIM), lambda b: (0, 0)),
        ],
        out_specs=[
            pl.BlockSpec((BLK, NE), lambda b: (b, 0)),
            pl.BlockSpec((BLK, NE), lambda b: (b, 0)),
            pl.BlockSpec((BLK, 1), lambda b: (b, 0)),
            pl.BlockSpec((BLK, 1), lambda b: (b, 0)),
        ],
        out_shape=[
            jax.ShapeDtypeStruct((T, NE), jnp.float32),
            jax.ShapeDtypeStruct((T, NE), jnp.float32),
            jax.ShapeDtypeStruct((T, 1), jnp.float32),
            jax.ShapeDtypeStruct((T, 1), jnp.float32),
        ],
    )


def _r2_body(m1_ref, m2_ref, idx_ref, be_ref, d1_ref, d2_ref, pincl_ref):
    tri_i = (_fiota((BLK, BLK), 0)
             >= _fiota((BLK, BLK), 1))
    tri_incl = tri_i.astype(jnp.float32)

    def cum_chunk(c, off):
        blk = m1_ref[pl.ds(c * BLK, BLK), :] + m2_ref[pl.ds(c * BLK, BLK), :]
        loc = lax.dot_general(tri_incl, blk, (((1,), (0,)), ((), ())),
                              precision=_HI,
                              preferred_element_type=jnp.float32)
        pincl_ref[pl.ds(c * BLK, BLK), :] = loc + off
        return off + loc[BLK - 1:BLK, :]

    total = lax.fori_loop(0, TB, cum_chunk, jnp.zeros((1, NE), jnp.float32))
    m1 = m1_ref[...]
    m2 = m2_ref[...]
    pincl = pincl_ref[...]
    pexcl = pincl - (m1 + m2)

    counts = total
    padded = jnp.ceil(counts * (1.0 / BLK)) * BLK
    tri8 = (_fiota((NE, NE), 0)
            < _fiota((NE, NE), 1)).astype(jnp.float32)
    o = lax.dot_general(padded, tri8, (((1,), (0,)), ((), ())),
                        precision=_HI,
                        preferred_element_type=jnp.float32)

    rank1 = jnp.sum(m1 * pexcl, axis=1, keepdims=True)
    rank2 = jnp.sum(m2 * pexcl, axis=1, keepdims=True)
    d1_ref[...] = (jnp.sum(m1 * o, axis=1, keepdims=True) + rank1).astype(jnp.int32)
    d2_ref[...] = (jnp.sum(m2 * o, axis=1, keepdims=True) + rank2).astype(jnp.int32)

    bids = _fiota((NB, 1), 0) * BLK
    ocols = o[:, 1:NE]
    be = jnp.sum((bids >= ocols).astype(jnp.float32), axis=1, keepdims=True)
    be_ref[...] = be.astype(jnp.int32)

    iota8r = _fiota((CHUNK, NE), 1)

    def src_chunk(c, _):
        p = (_fiota((CHUNK, 1), 0) + c * CHUNK)
        e = jnp.sum((p >= ocols).astype(jnp.float32), axis=1, keepdims=True)
        onehot = (iota8r == e).astype(jnp.float32)
        r = p - jnp.sum(onehot * o, axis=1, keepdims=True)
        csel = lax.dot_general(onehot, pincl, (((1,), (1,)), ((), ())),
                               precision=_HI,
                               preferred_element_type=jnp.float32)
        src = jnp.sum((csel <= r).astype(jnp.float32), axis=1, keepdims=True)
        src = jnp.minimum(src, float(T - 1))
        idx_ref[pl.ds(c * CHUNK, CHUNK), :] = src.astype(jnp.int32)
        return 0

    lax.fori_loop(0, NCHUNK, src_chunk, 0)


def _router_meta(m1, m2):
    return pl.pallas_call(
        _r2_body,
        out_shape=[
            jax.ShapeDtypeStruct((R, 1), jnp.int32),
            jax.ShapeDtypeStruct((NB, 1), jnp.int32),
            jax.ShapeDtypeStruct((T, 1), jnp.int32),
            jax.ShapeDtypeStruct((T, 1), jnp.int32),
        ],
        scratch_shapes=[pltpu.VMEM((T, NE), jnp.float32)],
    )(m1, m2)


def _sc_gather(table, idx, n_rows):
    info = plsc.get_sparse_core_info()
    nw = info.num_cores * info.num_subcores
    b_per_w = n_rows // nw
    ch = 16
    mesh = plsc.VectorSubcoreMesh(core_axis_name="c", subcore_axis_name="s")

    @functools.partial(
        pl.kernel, mesh=mesh,
        out_type=jax.ShapeDtypeStruct((n_rows, DIM), jnp.float32),
        scratch_types=[
            pltpu.VMEM((b_per_w,), jnp.int32),
            pltpu.VMEM((ch, DIM), jnp.float32),
            pltpu.SemaphoreType.DMA,
        ],
    )
    def k(table_hbm, idx_hbm, out_hbm, idx_v, rows_v, sem):
        wid = lax.axis_index("s") * info.num_cores + lax.axis_index("c")
        base = wid * b_per_w
        pltpu.sync_copy(idx_hbm.at[pl.ds(base, b_per_w)], idx_v)

        def body(j, _):
            pltpu.async_copy(table_hbm.at[idx_v.at[pl.ds(j * ch, ch)]],
                             rows_v, sem).wait()
            pltpu.sync_copy(rows_v, out_hbm.at[pl.ds(base + j * ch, ch)])
            return 0

        lax.fori_loop(0, b_per_w // ch, body, 0)

    return k(table, idx)


def _u_body(be_ref, x_ref, wu_ref, h_ref):
    del be_ref
    h = lax.dot_general(x_ref[...], wu_ref[0], (((1,), (1,)), ((), ())),
                        precision=_DEF, preferred_element_type=jnp.float32)
    h = jnp.maximum(h, 0.0)
    h_ref[...] = h * h


def _grouped_up(be, routed, w_up):
    return pl.pallas_call(
        _u_body,
        grid_spec=pltpu.PrefetchScalarGridSpec(
            num_scalar_prefetch=1,
            grid=(NB,),
            in_specs=[
                pl.BlockSpec((BLK, DIM), lambda b, be: (b, 0)),
                pl.BlockSpec((1, HID, DIM), lambda b, be: (be[b], 0, 0)),
            ],
            out_specs=pl.BlockSpec((BLK, HID), lambda b, be: (b, 0)),
        ),
        out_shape=jax.ShapeDtypeStruct((R, HID), jnp.float32),
    )(be, routed, w_up)


def _d_body(be_ref, h_ref, wd_ref, o_ref):
    del be_ref
    o_ref[...] = lax.dot_general(h_ref[...], wd_ref[0], (((1,), (1,)), ((), ())),
                                 precision=_DEF, preferred_element_type=jnp.float32)


def _grouped_down(be, h, w_down):
    return pl.pallas_call(
        _d_body,
        grid_spec=pltpu.PrefetchScalarGridSpec(
            num_scalar_prefetch=1,
            grid=(NB,),
            in_specs=[
                pl.BlockSpec((BLK, HID), lambda b, be: (b, 0)),
                pl.BlockSpec((1, DIM, HID), lambda b, be: (be[b], 0, 0)),
            ],
            out_specs=pl.BlockSpec((BLK, DIM), lambda b, be: (b, 0)),
        ),
        out_shape=jax.ShapeDtypeStruct((R, DIM), jnp.float32),
    )(be, h, w_down)


def _us_body(x_ref, wu_ref, h_ref):
    h = lax.dot_general(x_ref[...], wu_ref[...], (((1,), (1,)), ((), ())),
                        precision=_DEF, preferred_element_type=jnp.float32)
    h = jnp.maximum(h, 0.0)
    h_ref[...] = h * h


def _shared_up(x2, w_up_shared):
    return pl.pallas_call(
        _us_body,
        grid=(TB,),
        in_specs=[
            pl.BlockSpec((BLK, DIM), lambda b: (b, 0)),
            pl.BlockSpec((HID, DIM), lambda b: (0, 0)),
        ],
        out_specs=pl.BlockSpec((BLK, HID), lambda b: (b, 0)),
        out_shape=jax.ShapeDtypeStruct((T, HID), jnp.float32),
    )(x2, w_up_shared)


def _ds_body(hs_ref, wd_ref, a_ref, b_ref, w1_ref, w2_ref, o_ref):
    o = lax.dot_general(hs_ref[...], wd_ref[...], (((1,), (1,)), ((), ())),
                        precision=_DEF, preferred_element_type=jnp.float32)
    o_ref[...] = o + w1_ref[...] * a_ref[...] + w2_ref[...] * b_ref[...]


def _shared_down_combine(hs, w_down_shared, ab, w1, w2):
    return pl.pallas_call(
        _ds_body,
        grid=(TB,),
        in_specs=[
            pl.BlockSpec((BLK, HID), lambda b: (b, 0)),
            pl.BlockSpec((DIM, HID), lambda b: (0, 0)),
            pl.BlockSpec((BLK, DIM), lambda b: (b, 0)),
            pl.BlockSpec((BLK, DIM), lambda b: (b + TB, 0)),
            pl.BlockSpec((BLK, 1), lambda b: (b, 0)),
            pl.BlockSpec((BLK, 1), lambda b: (b, 0)),
        ],
        out_specs=pl.BlockSpec((BLK, DIM), lambda b: (b, 0)),
        out_shape=jax.ShapeDtypeStruct((T, DIM), jnp.float32),
    )(hs, w_down_shared, ab, ab, w1, w2)


def kernel(x, w_gate, w_up, w_down, w_up_shared, w_down_shared):
    x2 = x.reshape(T, DIM)
    m1, m2, w1, w2 = _router_select(x2)(x2, w_gate)
    idx, be, d1, d2 = _router_meta(m1, m2)

    routed = _sc_gather(x2, idx.reshape(R), R)
    h = _grouped_up(be.reshape(NB), routed, w_up)
    rows = _grouped_down(be.reshape(NB), h, w_down)

    d12 = jnp.concatenate([d1.reshape(T), d2.reshape(T)])
    ab = _sc_gather(rows, d12, 2 * T)

    hs = _shared_up(x2, w_up_shared)
    out = _shared_down_combine(hs, w_down_shared, ab, w1, w2)
    return out.reshape(1, T, DIM)
